# Initial kernel scaffold; baseline (speedup 1.0000x reference)
import functools

import jax
import jax.numpy as jnp
from jax.experimental import pallas as pl

S = 64.0
MARGIN = 0.5

_BLOCK_COLS = 1024


def _arcface_block(logits_ref, labels_ref, out_ref, *, block_cols):
    j = pl.program_id(0)
    x = logits_ref[...]
    rows, cols = x.shape
    col_ids = jax.lax.broadcasted_iota(jnp.int32, (rows, cols), 1) + j * block_cols
    mask = col_ids == labels_ref[...]
    t = jnp.sum(jnp.where(mask, x, 0.0), axis=1, keepdims=True)
    fixed = jnp.cos(jnp.arccos(t) + MARGIN) * S
    out_ref[...] = jnp.where(mask, fixed, x * S)


def kernel(logits, labels):
    n_rows, n_cols = logits.shape
    grid = (pl.cdiv(n_cols, _BLOCK_COLS),)
    labels2d = labels.reshape(n_rows, 1)
    return pl.pallas_call(
        functools.partial(_arcface_block, block_cols=_BLOCK_COLS),
        grid=grid,
        in_specs=[
            pl.BlockSpec((n_rows, _BLOCK_COLS), lambda j: (0, j)),
            pl.BlockSpec((n_rows, 1), lambda j: (0, 0)),
        ],
        out_specs=pl.BlockSpec((n_rows, _BLOCK_COLS), lambda j: (0, j)),
        out_shape=jax.ShapeDtypeStruct((n_rows, n_cols), jnp.float32),
    )(logits, labels2d)


# speedup vs baseline: 2.8987x; 2.8987x over previous
import functools
import math

import jax
import jax.numpy as jnp
from jax.experimental import pallas as pl

S = 64.0
MARGIN = 0.5
_COS_M = math.cos(MARGIN)
_SIN_M = math.sin(MARGIN)

_BLOCK_COLS = 1024


def _arcface_block(logits_ref, labels_ref, out_ref, *, block_cols):
    j = pl.program_id(0)
    x = logits_ref[...]
    rows, cols = x.shape
    col_ids = jax.lax.broadcasted_iota(jnp.int32, (rows, cols), 1) + j * block_cols
    mask = col_ids == labels_ref[...]
    t = jnp.sum(jnp.where(mask, x, 0.0), axis=1, keepdims=True)
    sin_t = jnp.sqrt(jnp.maximum(1.0 - t * t, 0.0))
    fixed = (t * _COS_M - sin_t * _SIN_M) * S
    out_ref[...] = jnp.where(mask, fixed, x * S)


def kernel(logits, labels):
    n_rows, n_cols = logits.shape
    grid = (pl.cdiv(n_cols, _BLOCK_COLS),)
    labels2d = labels.reshape(n_rows, 1)
    return pl.pallas_call(
        functools.partial(_arcface_block, block_cols=_BLOCK_COLS),
        grid=grid,
        in_specs=[
            pl.BlockSpec((n_rows, _BLOCK_COLS), lambda j: (0, j)),
            pl.BlockSpec((n_rows, 1), lambda j: (0, 0)),
        ],
        out_specs=pl.BlockSpec((n_rows, _BLOCK_COLS), lambda j: (0, j)),
        out_shape=jax.ShapeDtypeStruct((n_rows, n_cols), jnp.float32),
    )(logits, labels2d)


# speedup vs baseline: 2.9188x; 1.0069x over previous
import functools
import math

import jax
import jax.numpy as jnp
from jax.experimental import pallas as pl

S = 64.0
MARGIN = 0.5
_COS_M = math.cos(MARGIN)
_SIN_M = math.sin(MARGIN)

_BLOCK_COLS = 2048


def _arcface_block(logits_ref, labels_ref, out_ref, *, block_cols):
    j = pl.program_id(0)
    x = logits_ref[...]
    rows, cols = x.shape
    col_ids = jax.lax.broadcasted_iota(jnp.int32, (rows, cols), 1) + j * block_cols
    mask = col_ids == labels_ref[...]
    t = jnp.sum(jnp.where(mask, x, 0.0), axis=1, keepdims=True)
    sin_t = jnp.sqrt(jnp.maximum(1.0 - t * t, 0.0))
    fixed = (t * _COS_M - sin_t * _SIN_M) * S
    out_ref[...] = jnp.where(mask, fixed, x * S)


def kernel(logits, labels):
    n_rows, n_cols = logits.shape
    grid = (pl.cdiv(n_cols, _BLOCK_COLS),)
    labels2d = labels.reshape(n_rows, 1)
    return pl.pallas_call(
        functools.partial(_arcface_block, block_cols=_BLOCK_COLS),
        grid=grid,
        in_specs=[
            pl.BlockSpec((n_rows, _BLOCK_COLS), lambda j: (0, j)),
            pl.BlockSpec((n_rows, 1), lambda j: (0, 0)),
        ],
        out_specs=pl.BlockSpec((n_rows, _BLOCK_COLS), lambda j: (0, j)),
        out_shape=jax.ShapeDtypeStruct((n_rows, n_cols), jnp.float32),
    )(logits, labels2d)
